# Initial kernel scaffold; baseline (speedup 1.0000x reference)
import jax
import jax.numpy as jnp
from jax.experimental import pallas as pl


def kernel(pos_enc, x):
    raise NotImplementedError("write your pallas kernel here")


# speedup vs baseline: 1.4800x; 1.4800x over previous
import functools

import jax
import jax.numpy as jnp
from jax import lax
from jax.experimental import pallas as pl
from jax.experimental.pallas import tpu as pltpu
from jax.experimental.pallas import tpu_sc as plsc

MODEL_DIM = 4096
MAXLEN = 2048
ROWS = 4 * 2048
NUM_CORES = 2
NUM_SUBCORES = 16
NW = NUM_CORES * NUM_SUBCORES
RPW = ROWS // NW
CH = 8
NCH = RPW // CH

_mesh = plsc.VectorSubcoreMesh(core_axis_name="c", subcore_axis_name="s")


@functools.partial(
    pl.kernel,
    out_type=jax.ShapeDtypeStruct((ROWS, MODEL_DIM), jnp.float32),
    mesh=_mesh,
    scratch_types=[
        pltpu.VMEM((NCH, CH), jnp.int32),
        pltpu.VMEM((CH, MODEL_DIM), jnp.float32),
        pltpu.SemaphoreType.DMA,
    ],
)
def _gather_rows(table, idx, out, idx_v, buf, sem):
    wid = lax.axis_index("s") * NUM_CORES + lax.axis_index("c")
    base = wid * RPW
    pltpu.sync_copy(idx.at[wid], idx_v)

    def body(c, _):
        pltpu.async_copy(table.at[idx_v.at[c]], buf, sem).wait()
        pltpu.sync_copy(buf, out.at[pl.ds(base + c * CH, CH)])
        return 0

    lax.fori_loop(0, NCH, body, 0)


def kernel(pos_enc, x):
    idx = x.reshape(NW, NCH, CH).astype(jnp.int32)
    out = _gather_rows(pos_enc, idx)
    return out.reshape(x.shape[0], x.shape[1], MODEL_DIM)


# speedup vs baseline: 1.6792x; 1.1346x over previous
import functools

import jax
import jax.numpy as jnp
from jax import lax
from jax.experimental import pallas as pl
from jax.experimental.pallas import tpu as pltpu
from jax.experimental.pallas import tpu_sc as plsc

MODEL_DIM = 4096
MAXLEN = 2048
ROWS = 4 * 2048
NUM_CORES = 2
NUM_SUBCORES = 16
NW = NUM_CORES * NUM_SUBCORES
RPW = ROWS // NW
CH = 8
NCH = RPW // CH

_mesh = plsc.VectorSubcoreMesh(core_axis_name="c", subcore_axis_name="s")


@functools.partial(
    pl.kernel,
    out_type=jax.ShapeDtypeStruct((ROWS, MODEL_DIM), jnp.float32),
    mesh=_mesh,
    scratch_types=[
        pltpu.VMEM((NCH, CH), jnp.int32),
        pltpu.VMEM((CH, MODEL_DIM), jnp.float32),
        pltpu.VMEM((CH, MODEL_DIM), jnp.float32),
        pltpu.SemaphoreType.DMA,
        pltpu.SemaphoreType.DMA,
        pltpu.SemaphoreType.DMA,
        pltpu.SemaphoreType.DMA,
    ],
)
def _gather_rows(table, idx, out, idx_v, buf0, buf1, gsem0, gsem1, ssem0, ssem1):
    wid = lax.axis_index("s") * NUM_CORES + lax.axis_index("c")
    base = wid * RPW
    pltpu.sync_copy(idx.at[wid], idx_v)

    g0 = pltpu.async_copy(table.at[idx_v.at[0]], buf0, gsem0)
    g1 = pltpu.async_copy(table.at[idx_v.at[1]], buf1, gsem1)
    ngroups = NCH // 2

    def body(g, _):
        c0 = 2 * g
        g0.wait()
        s0 = pltpu.async_copy(buf0, out.at[pl.ds(base + c0 * CH, CH)], ssem0)
        g1.wait()
        s1 = pltpu.async_copy(buf1, out.at[pl.ds(base + (c0 + 1) * CH, CH)], ssem1)

        @pl.when(g + 1 < ngroups)
        def _():
            s0.wait()
            pltpu.async_copy(table.at[idx_v.at[c0 + 2]], buf0, gsem0)
            s1.wait()
            pltpu.async_copy(table.at[idx_v.at[c0 + 3]], buf1, gsem1)

        return 0

    lax.fori_loop(0, ngroups, body, 0)
    pltpu.make_async_copy(buf0, out.at[pl.ds(base + (RPW - 2 * CH), CH)], ssem0).wait()
    pltpu.make_async_copy(buf1, out.at[pl.ds(base + (RPW - CH), CH)], ssem1).wait()


def kernel(pos_enc, x):
    idx = x.reshape(NW, NCH, CH).astype(jnp.int32)
    out = _gather_rows(pos_enc, idx)
    return out.reshape(x.shape[0], x.shape[1], MODEL_DIM)


# speedup vs baseline: 1.7817x; 1.0611x over previous
import functools

import jax
import jax.numpy as jnp
from jax import lax
from jax.experimental import pallas as pl
from jax.experimental.pallas import tpu as pltpu
from jax.experimental.pallas import tpu_sc as plsc

MODEL_DIM = 4096
MAXLEN = 2048
ROWS = 4 * 2048
NUM_CORES = 2
NUM_SUBCORES = 16
NW = NUM_CORES * NUM_SUBCORES
RPW = ROWS // NW
CH = 8
NCH = RPW // CH
NBUF = 3
NGRP = NCH // NBUF
TAIL = NCH - NGRP * NBUF

_mesh = plsc.VectorSubcoreMesh(core_axis_name="c", subcore_axis_name="s")


@functools.partial(
    pl.kernel,
    out_type=jax.ShapeDtypeStruct((ROWS, MODEL_DIM), jnp.float32),
    mesh=_mesh,
    scratch_types=[
        pltpu.VMEM((NCH, CH), jnp.int32),
        [pltpu.VMEM((CH, MODEL_DIM), jnp.float32) for _ in range(NBUF)],
        [pltpu.SemaphoreType.DMA for _ in range(NBUF)],
        [pltpu.SemaphoreType.DMA for _ in range(NBUF)],
    ],
)
def _gather_rows(table, idx, out, idx_v, bufs, gsems, ssems):
    wid = lax.axis_index("s") * NUM_CORES + lax.axis_index("c")
    base = wid * RPW
    pltpu.sync_copy(idx.at[wid], idx_v)

    def gather(c, b):
        pltpu.async_copy(table.at[idx_v.at[c]], bufs[b], gsems[b])

    def store(c, b):
        pltpu.async_copy(bufs[b], out.at[pl.ds(base + c * CH, CH)], ssems[b])

    def wait_g(b):
        pltpu.make_async_copy(table.at[idx_v.at[0]], bufs[b], gsems[b]).wait()

    def wait_s(b):
        pltpu.make_async_copy(bufs[b], out.at[pl.ds(base, CH)], ssems[b]).wait()

    for b in range(NBUF):
        gather(b, b)

    def body(g, _):
        c_base = g * NBUF
        for b in range(NBUF):
            c = c_base + b
            wait_g(b)
            store(c, b)

            @pl.when(c + NBUF < NCH)
            def _():
                wait_s(b)
                gather(c + NBUF, b)

        return 0

    lax.fori_loop(0, NGRP, body, 0)

    for b in range(TAIL):
        wait_g(b)
        store(NGRP * NBUF + b, b)
    for b in range(NBUF):
        wait_s(b)


def kernel(pos_enc, x):
    idx = x.reshape(NW, NCH, CH).astype(jnp.int32)
    out = _gather_rows(pos_enc, idx)
    return out.reshape(x.shape[0], x.shape[1], MODEL_DIM)
